# baseline (speedup 1.0000x reference)
import functools

import jax
import jax.numpy as jnp
from jax import lax
from jax.experimental import pallas as pl
from jax.experimental.pallas import tpu as pltpu
from jax.experimental.pallas import tpu_sc as plsc

Q = 1024
D = 16
K_REAL = 100000
BLK = 1024
NBLK = 98
K_PAD = NBLK * BLK
NDOCS = 10
CHUNK = 128
CPB = BLK // CHUNK
NCHUNK = NBLK * CPB
CID_W = 16 + NDOCS * 16
NEG = float("-inf")

NWORKERS = 32
NSPLIT = 2
QH = Q // NSPLIT
QPW = QH // NWORKERS


def _tc_kernel(q_ref, k_ref, scores_ref, cids_ref, cm_ref):
    b = pl.program_id(0)
    qm = q_ref[...]
    kb = k_ref[...]
    s = lax.dot_general(
        kb, qm, (((1,), (1,)), ((), ())),
        preferred_element_type=jnp.float32)
    row = lax.broadcasted_iota(jnp.int32, (BLK, QH), 0)
    s = jnp.where(row + b * BLK < K_REAL, s, NEG)

    for c in range(CPB):
        scores_ref[c] = s[c * CHUNK:(c + 1) * CHUNK, :].T

    cm = jnp.concatenate(
        [jnp.max(s[c * CHUNK:(c + 1) * CHUNK, :], axis=0, keepdims=True)
         for c in range(CPB)], axis=0)
    cm_ref[pl.ds(b * CPB, CPB), :] = cm

    @pl.when(b == NBLK - 1)
    def _pick_chunks():
        cmv = cm_ref[...]
        crow = lax.broadcasted_iota(jnp.int32, (NCHUNK, QH), 0)
        ids = []
        for _ in range(NDOCS):
            m = jnp.max(cmv, axis=0, keepdims=True)
            al = jnp.min(jnp.where(cmv == m, crow, NCHUNK), axis=0,
                         keepdims=True)
            ids.append(al)
            cmv = jnp.where(crow == al, NEG, cmv)
        packed = jnp.concatenate(
            ids + [jnp.zeros((16 - NDOCS, QH), jnp.int32)], axis=0)
        splats = [jnp.broadcast_to(ids[r], (16, QH)) for r in range(NDOCS)]
        cids_ref[...] = jnp.concatenate([packed] + splats, axis=0).T


def _sc_kernel(scores_hbm, cids_hbm, keys_hbm, vals_hbm, idx_hbm, sel_hbm,
               cids_v, gidx_a, gidx_b, buf_a, buf_b, ovals_v, oidx_v,
               selidx_v, selvec_v, sem_a, sem_b):
    wid = lax.axis_index("s") * 2 + lax.axis_index("c")
    base = wid * QPW
    pltpu.sync_copy(cids_hbm.at[pl.ds(base, QPW)], cids_v)
    iota16 = lax.iota(jnp.int32, 16)

    def fire(qi, gref, bref, sem):
        gref[...] = cids_v[qi, 0:16] * QH + (base + qi)
        pltpu.async_copy(scores_hbm.at[gref], bref, sem)

    def drain(gref, bref, sem):
        pltpu.make_async_copy(scores_hbm.at[gref], bref, sem).wait()

    def process(qi, bref):
        accs = [(jnp.full((16,), NEG, jnp.float32),
                 jnp.zeros((16,), jnp.int32)) for _ in range(CHUNK // 16)]
        for r in range(NDOCS):
            csplat = cids_v[qi, 16 + r * 16:32 + r * 16]
            cbase = csplat * CHUNK
            for v in range(CHUNK // 16):
                cand_v, cand_i = accs[v]
                vv = bref[r, v * 16:(v + 1) * 16]
                gi = cbase + v * 16 + iota16
                sv, si = plsc.sort_key_val(vv, gi, descending=True)
                take = sv > cand_v
                accs[v] = plsc.sort_key_val(
                    jnp.where(take, sv, cand_v),
                    jnp.where(take, si, cand_i))

        def merge(a, b):
            bv = lax.rev(b[0], (0,))
            bi = lax.rev(b[1], (0,))
            take = bv > a[0]
            return plsc.sort_key_val(jnp.where(take, bv, a[0]),
                                     jnp.where(take, bi, a[1]))

        while len(accs) > 1:
            accs = [merge(accs[i], accs[i + 1])
                    for i in range(0, len(accs), 2)]
        cand_v, cand_i = accs[0]
        rv = lax.rev(cand_v, (0,))
        ri = lax.rev(cand_i, (0,))
        ovals_v[qi, :] = rv
        oidx_v[qi, :] = ri
        plsc.store_scatter(selidx_v, [jnp.full((16,), qi, jnp.int32)], ri,
                           mask=iota16 == 0)

    fire(0, gidx_a, buf_a, sem_a)

    def pair(i, carry):
        q0 = 2 * i
        q1 = q0 + 1
        fire(q1, gidx_b, buf_b, sem_b)
        drain(gidx_a, buf_a, sem_a)
        process(q0, buf_a)
        fire(jnp.minimum(q0 + 2, QPW - 1), gidx_a, buf_a, sem_a)
        drain(gidx_b, buf_b, sem_b)
        process(q1, buf_b)
        return carry

    lax.fori_loop(0, QPW // 2, pair, 0)
    drain(gidx_a, buf_a, sem_a)

    pltpu.sync_copy(ovals_v, vals_hbm.at[pl.ds(base, QPW)])
    pltpu.sync_copy(oidx_v, idx_hbm.at[pl.ds(base, QPW)])
    pltpu.async_copy(keys_hbm.at[selidx_v], selvec_v, sem_a).wait()
    pltpu.sync_copy(selvec_v, sel_hbm.at[pl.ds(base, QPW)])


def _run_half(queries_h, keys_p, keys128):
    scores, cids = pl.pallas_call(
        _tc_kernel,
        grid=(NBLK,),
        in_specs=[
            pl.BlockSpec((QH, D), lambda b: (0, 0)),
            pl.BlockSpec((BLK, D), lambda b: (b, 0)),
        ],
        out_specs=[
            pl.BlockSpec((CPB, QH, CHUNK), lambda b: (b, 0, 0)),
            pl.BlockSpec((QH, CID_W), lambda b: (0, 0)),
        ],
        out_shape=[
            jax.ShapeDtypeStruct((NCHUNK, QH, CHUNK), jnp.float32),
            jax.ShapeDtypeStruct((QH, CID_W), jnp.int32),
        ],
        scratch_shapes=[
            pltpu.VMEM((NCHUNK, QH), jnp.float32),
        ],
    )(queries_h, keys_p)

    scores2d = scores.reshape(NCHUNK * QH, CHUNK)

    mesh = plsc.VectorSubcoreMesh(core_axis_name="c", subcore_axis_name="s")
    vals, idx, sel = pl.kernel(
        _sc_kernel,
        mesh=mesh,
        compiler_params=pltpu.CompilerParams(needs_layout_passes=False),
        out_type=[
            jax.ShapeDtypeStruct((QH, 16), jnp.float32),
            jax.ShapeDtypeStruct((QH, 16), jnp.int32),
            jax.ShapeDtypeStruct((QH, 128), jnp.float32),
        ],
        scratch_types=[
            pltpu.VMEM((QPW, CID_W), jnp.int32),
            pltpu.VMEM((16,), jnp.int32),
            pltpu.VMEM((16,), jnp.int32),
            pltpu.VMEM((16, CHUNK), jnp.float32),
            pltpu.VMEM((16, CHUNK), jnp.float32),
            pltpu.VMEM((QPW, 16), jnp.float32),
            pltpu.VMEM((QPW, 16), jnp.int32),
            pltpu.VMEM((QPW,), jnp.int32),
            pltpu.VMEM((QPW, 128), jnp.float32),
            pltpu.SemaphoreType.DMA,
            pltpu.SemaphoreType.DMA,
        ],
    )(scores2d, cids, keys128)
    return vals, idx, sel


@jax.jit
def kernel(queries, keys):
    keys_p = jnp.pad(keys, ((0, K_PAD - K_REAL), (0, 0)))
    keys128 = jnp.pad(keys_p, ((0, 0), (0, 128 - D)))

    outs = [_run_half(queries[i * QH:(i + 1) * QH], keys_p, keys128)
            for i in range(NSPLIT)]
    vals = jnp.concatenate([o[0] for o in outs], axis=0)
    idx = jnp.concatenate([o[1] for o in outs], axis=0)
    sel = jnp.concatenate([o[2] for o in outs], axis=0)
    return vals[:, :NDOCS], idx[:, :NDOCS], sel[:, :D]


# speedup vs baseline: 1.1909x; 1.1909x over previous
import jax
import jax.numpy as jnp
from jax import lax
from jax.experimental import pallas as pl
from jax.experimental.pallas import tpu as pltpu
from jax.experimental.pallas import tpu_sc as plsc

Q = 1024
D = 16
K_REAL = 100000
BLK = 1024
NBLK = 98
K_PAD = NBLK * BLK
NDOCS = 10
CHUNK = 128
CPB = BLK // CHUNK
NCHUNK = NBLK * CPB
CID_W = 16 + NDOCS * 16
NEG = float("-inf")

NWORKERS = 32
QPW = Q // NWORKERS


def _tc_kernel(q_ref, k_ref, scores_ref, cids_ref, cm_ref):
    b = pl.program_id(0)
    qm = q_ref[...]
    kb = k_ref[...]
    s = lax.dot_general(
        kb, qm, (((1,), (1,)), ((), ())),
        preferred_element_type=jnp.float32)
    row = lax.broadcasted_iota(jnp.int32, (BLK, Q), 0)
    s = jnp.where(row + b * BLK < K_REAL, s, NEG)

    for c in range(CPB):
        scores_ref[c] = s[c * CHUNK:(c + 1) * CHUNK, :].T

    cm = jnp.concatenate(
        [jnp.max(s[c * CHUNK:(c + 1) * CHUNK, :], axis=0, keepdims=True)
         for c in range(CPB)], axis=0)
    cm_ref[pl.ds(b * CPB, CPB), :] = cm

    @pl.when(b == NBLK - 1)
    def _pick_chunks():
        cmv = cm_ref[...]
        crow = lax.broadcasted_iota(jnp.int32, (NCHUNK, Q), 0)
        ids = []
        for _ in range(NDOCS):
            m = jnp.max(cmv, axis=0, keepdims=True)
            al = jnp.min(jnp.where(cmv == m, crow, NCHUNK), axis=0,
                         keepdims=True)
            ids.append(al)
            cmv = jnp.where(crow == al, NEG, cmv)
        packed = jnp.concatenate(
            ids + [jnp.zeros((16 - NDOCS, Q), jnp.int32)], axis=0)
        splats = [jnp.broadcast_to(ids[r], (16, Q)) for r in range(NDOCS)]
        cids_ref[...] = jnp.concatenate([packed] + splats, axis=0).T


def _sc_kernel(scores_hbm, cids_hbm, keys_hbm, vals_hbm, idx_hbm, sel_hbm,
               cids_v, gidx_a, gidx_b, buf_a, buf_b, ovals_v, oidx_v,
               selidx_v, selvec_v, sem_a, sem_b):
    wid = lax.axis_index("s") * 2 + lax.axis_index("c")
    base = wid * QPW
    pltpu.sync_copy(cids_hbm.at[pl.ds(base, QPW)], cids_v)
    iota16 = lax.iota(jnp.int32, 16)

    def fire(qi, gref, bref, sem):
        gref[...] = cids_v[qi, 0:16] * Q + (base + qi)
        pltpu.async_copy(scores_hbm.at[gref], bref, sem)

    def drain(gref, bref, sem):
        pltpu.make_async_copy(scores_hbm.at[gref], bref, sem).wait()

    def process(qi, bref):
        accs = [(jnp.full((16,), NEG, jnp.float32),
                 jnp.zeros((16,), jnp.int32)) for _ in range(CHUNK // 16)]
        for r in range(NDOCS):
            csplat = cids_v[qi, 16 + r * 16:32 + r * 16]
            cbase = csplat * CHUNK
            for v in range(CHUNK // 16):
                cand_v, cand_i = accs[v]
                vv = bref[r, v * 16:(v + 1) * 16]
                gi = cbase + v * 16 + iota16
                sv, si = plsc.sort_key_val(vv, gi, descending=True)
                take = sv > cand_v
                accs[v] = plsc.sort_key_val(
                    jnp.where(take, sv, cand_v),
                    jnp.where(take, si, cand_i))

        def merge(a, b):
            bv = lax.rev(b[0], (0,))
            bi = lax.rev(b[1], (0,))
            take = bv > a[0]
            return plsc.sort_key_val(jnp.where(take, bv, a[0]),
                                     jnp.where(take, bi, a[1]))

        while len(accs) > 1:
            accs = [merge(accs[i], accs[i + 1])
                    for i in range(0, len(accs), 2)]
        cand_v, cand_i = accs[0]
        rv = lax.rev(cand_v, (0,))
        ri = lax.rev(cand_i, (0,))
        ovals_v[qi, :] = rv
        oidx_v[qi, :] = ri
        plsc.store_scatter(selidx_v, [jnp.full((16,), qi, jnp.int32)], ri,
                           mask=iota16 == 0)

    fire(0, gidx_a, buf_a, sem_a)

    def pair(i, carry):
        q0 = 2 * i
        q1 = q0 + 1
        fire(q1, gidx_b, buf_b, sem_b)
        drain(gidx_a, buf_a, sem_a)
        process(q0, buf_a)
        fire(jnp.minimum(q0 + 2, QPW - 1), gidx_a, buf_a, sem_a)
        drain(gidx_b, buf_b, sem_b)
        process(q1, buf_b)
        return carry

    lax.fori_loop(0, QPW // 2, pair, 0)
    drain(gidx_a, buf_a, sem_a)

    pltpu.sync_copy(ovals_v, vals_hbm.at[pl.ds(base, QPW)])
    pltpu.sync_copy(oidx_v, idx_hbm.at[pl.ds(base, QPW)])
    pltpu.async_copy(keys_hbm.at[selidx_v], selvec_v, sem_a).wait()
    pltpu.sync_copy(selvec_v, sel_hbm.at[pl.ds(base, QPW)])


@jax.jit
def kernel(queries, keys):
    keys_p = jnp.pad(keys, ((0, K_PAD - K_REAL), (0, 0)))

    scores, cids = pl.pallas_call(
        _tc_kernel,
        grid=(NBLK,),
        in_specs=[
            pl.BlockSpec((Q, D), lambda b: (0, 0)),
            pl.BlockSpec((BLK, D), lambda b: (b, 0)),
        ],
        out_specs=[
            pl.BlockSpec((CPB, Q, CHUNK), lambda b: (b, 0, 0)),
            pl.BlockSpec((Q, CID_W), lambda b: (0, 0)),
        ],
        out_shape=[
            jax.ShapeDtypeStruct((NCHUNK, Q, CHUNK), jnp.float32),
            jax.ShapeDtypeStruct((Q, CID_W), jnp.int32),
        ],
        scratch_shapes=[
            pltpu.VMEM((NCHUNK, Q), jnp.float32),
        ],
    )(queries, keys_p)

    scores2d = scores.reshape(NCHUNK * Q, CHUNK)

    mesh = plsc.VectorSubcoreMesh(core_axis_name="c", subcore_axis_name="s")
    keys128 = jnp.pad(keys_p, ((0, 0), (0, 128 - D)))

    vals, idx, sel = pl.kernel(
        _sc_kernel,
        mesh=mesh,
        compiler_params=pltpu.CompilerParams(needs_layout_passes=False),
        out_type=[
            jax.ShapeDtypeStruct((Q, 16), jnp.float32),
            jax.ShapeDtypeStruct((Q, 16), jnp.int32),
            jax.ShapeDtypeStruct((Q, 128), jnp.float32),
        ],
        scratch_types=[
            pltpu.VMEM((QPW, CID_W), jnp.int32),
            pltpu.VMEM((16,), jnp.int32),
            pltpu.VMEM((16,), jnp.int32),
            pltpu.VMEM((16, CHUNK), jnp.float32),
            pltpu.VMEM((16, CHUNK), jnp.float32),
            pltpu.VMEM((QPW, 16), jnp.float32),
            pltpu.VMEM((QPW, 16), jnp.int32),
            pltpu.VMEM((QPW,), jnp.int32),
            pltpu.VMEM((QPW, 128), jnp.float32),
            pltpu.SemaphoreType.DMA,
            pltpu.SemaphoreType.DMA,
        ],
    )(scores2d, cids, keys128)

    return vals[:, :NDOCS], idx[:, :NDOCS], sel[:, :D]


# speedup vs baseline: 1.2981x; 1.0900x over previous
import jax
import jax.numpy as jnp
from jax import lax
from jax.experimental import pallas as pl
from jax.experimental.pallas import tpu as pltpu
from jax.experimental.pallas import tpu_sc as plsc

Q = 1024
D = 16
K_REAL = 100000
BLK = 2048
NBLK = 49
K_PAD = NBLK * BLK
NDOCS = 10
CHUNK = 128
CPB = BLK // CHUNK
NCHUNK = NBLK * CPB
CID_W = 16 + NDOCS * 16
NEG = float("-inf")

NWORKERS = 32
QPW = Q // NWORKERS


def _tc_kernel(q_ref, k_ref, scores_ref, cids_ref, cm_ref):
    b = pl.program_id(0)
    qm = q_ref[...]
    kb = k_ref[...]
    s = lax.dot_general(
        kb, qm, (((1,), (1,)), ((), ())),
        preferred_element_type=jnp.float32)
    row = lax.broadcasted_iota(jnp.int32, (BLK, Q), 0)
    s = jnp.where(row + b * BLK < K_REAL, s, NEG)

    for c in range(CPB):
        scores_ref[c] = s[c * CHUNK:(c + 1) * CHUNK, :].T

    cm = jnp.concatenate(
        [jnp.max(s[c * CHUNK:(c + 1) * CHUNK, :], axis=0, keepdims=True)
         for c in range(CPB)], axis=0)
    cm_ref[pl.ds(b * CPB, CPB), :] = cm

    @pl.when(b == NBLK - 1)
    def _pick_chunks():
        cmv = cm_ref[...]
        crow = lax.broadcasted_iota(jnp.int32, (NCHUNK, Q), 0)
        ids = []
        for _ in range(NDOCS):
            m = jnp.max(cmv, axis=0, keepdims=True)
            al = jnp.min(jnp.where(cmv == m, crow, NCHUNK), axis=0,
                         keepdims=True)
            ids.append(al)
            cmv = jnp.where(crow == al, NEG, cmv)
        packed = jnp.concatenate(
            ids + [jnp.zeros((16 - NDOCS, Q), jnp.int32)], axis=0)
        splats = [jnp.broadcast_to(ids[r], (16, Q)) for r in range(NDOCS)]
        cids_ref[...] = jnp.concatenate([packed] + splats, axis=0).T


def _sc_kernel(scores_hbm, cids_hbm, keys_hbm, vals_hbm, idx_hbm, sel_hbm,
               cids_v, gidx_a, gidx_b, buf_a, buf_b, ovals_v, oidx_v,
               selidx_v, selvec_v, sem_a, sem_b):
    wid = lax.axis_index("s") * 2 + lax.axis_index("c")
    base = wid * QPW
    pltpu.sync_copy(cids_hbm.at[pl.ds(base, QPW)], cids_v)
    iota16 = lax.iota(jnp.int32, 16)

    def fire(qi, gref, bref, sem):
        gref[...] = cids_v[qi, 0:16] * Q + (base + qi)
        pltpu.async_copy(scores_hbm.at[gref], bref, sem)

    def drain(gref, bref, sem):
        pltpu.make_async_copy(scores_hbm.at[gref], bref, sem).wait()

    def process(qi, bref):
        cand_v = jnp.full((16,), NEG, jnp.float32)
        cand_i = jnp.zeros((16,), jnp.int32)
        for r in range(NDOCS):
            csplat = cids_v[qi, 16 + r * 16:32 + r * 16]
            cbase = csplat * CHUNK
            for v in range(CHUNK // 16):
                vv = bref[r, v * 16:(v + 1) * 16]
                gi = cbase + v * 16 + iota16
                sv, si = plsc.sort_key_val(vv, gi, descending=True)
                take = sv > cand_v
                cand_v, cand_i = plsc.sort_key_val(
                    jnp.where(take, sv, cand_v),
                    jnp.where(take, si, cand_i))
        rv = lax.rev(cand_v, (0,))
        ri = lax.rev(cand_i, (0,))
        ovals_v[qi, :] = rv
        oidx_v[qi, :] = ri
        plsc.store_scatter(selidx_v, [jnp.full((16,), qi, jnp.int32)], ri,
                           mask=iota16 == 0)

    fire(0, gidx_a, buf_a, sem_a)

    def pair(i, carry):
        q0 = 2 * i
        q1 = q0 + 1
        fire(q1, gidx_b, buf_b, sem_b)
        drain(gidx_a, buf_a, sem_a)
        process(q0, buf_a)
        fire(jnp.minimum(q0 + 2, QPW - 1), gidx_a, buf_a, sem_a)
        drain(gidx_b, buf_b, sem_b)
        process(q1, buf_b)
        return carry

    lax.fori_loop(0, QPW // 2, pair, 0)
    drain(gidx_a, buf_a, sem_a)

    pltpu.sync_copy(ovals_v, vals_hbm.at[pl.ds(base, QPW)])
    pltpu.sync_copy(oidx_v, idx_hbm.at[pl.ds(base, QPW)])
    pltpu.async_copy(keys_hbm.at[selidx_v], selvec_v, sem_a).wait()
    pltpu.sync_copy(selvec_v, sel_hbm.at[pl.ds(base, QPW)])


@jax.jit
def kernel(queries, keys):
    keys_p = jnp.pad(keys, ((0, K_PAD - K_REAL), (0, 0)))

    scores, cids = pl.pallas_call(
        _tc_kernel,
        grid=(NBLK,),
        in_specs=[
            pl.BlockSpec((Q, D), lambda b: (0, 0)),
            pl.BlockSpec((BLK, D), lambda b: (b, 0)),
        ],
        out_specs=[
            pl.BlockSpec((CPB, Q, CHUNK), lambda b: (b, 0, 0)),
            pl.BlockSpec((Q, CID_W), lambda b: (0, 0)),
        ],
        out_shape=[
            jax.ShapeDtypeStruct((NCHUNK, Q, CHUNK), jnp.float32),
            jax.ShapeDtypeStruct((Q, CID_W), jnp.int32),
        ],
        scratch_shapes=[
            pltpu.VMEM((NCHUNK, Q), jnp.float32),
        ],
    )(queries, keys_p)

    scores2d = scores.reshape(NCHUNK * Q, CHUNK)

    mesh = plsc.VectorSubcoreMesh(core_axis_name="c", subcore_axis_name="s")
    keys128 = jnp.pad(keys_p, ((0, 0), (0, 128 - D)))

    vals, idx, sel = pl.kernel(
        _sc_kernel,
        mesh=mesh,
        compiler_params=pltpu.CompilerParams(needs_layout_passes=False),
        out_type=[
            jax.ShapeDtypeStruct((Q, 16), jnp.float32),
            jax.ShapeDtypeStruct((Q, 16), jnp.int32),
            jax.ShapeDtypeStruct((Q, 128), jnp.float32),
        ],
        scratch_types=[
            pltpu.VMEM((QPW, CID_W), jnp.int32),
            pltpu.VMEM((16,), jnp.int32),
            pltpu.VMEM((16,), jnp.int32),
            pltpu.VMEM((16, CHUNK), jnp.float32),
            pltpu.VMEM((16, CHUNK), jnp.float32),
            pltpu.VMEM((QPW, 16), jnp.float32),
            pltpu.VMEM((QPW, 16), jnp.int32),
            pltpu.VMEM((QPW,), jnp.int32),
            pltpu.VMEM((QPW, 128), jnp.float32),
            pltpu.SemaphoreType.DMA,
            pltpu.SemaphoreType.DMA,
        ],
    )(scores2d, cids, keys128)

    return vals[:, :NDOCS], idx[:, :NDOCS], sel[:, :D]


# speedup vs baseline: 1.3029x; 1.0037x over previous
import jax
import jax.numpy as jnp
from jax import lax
from jax.experimental import pallas as pl
from jax.experimental.pallas import tpu as pltpu
from jax.experimental.pallas import tpu_sc as plsc

Q = 1024
D = 16
K_REAL = 100000
BLK = 4096
NBLK = 25
K_PAD = NBLK * BLK
NDOCS = 10
CHUNK = 128
CPB = BLK // CHUNK
NCHUNK = NBLK * CPB
CID_W = 16 + NDOCS * 16
NEG = float("-inf")

NWORKERS = 32
QPW = Q // NWORKERS


def _tc_kernel(q_ref, k_ref, scores_ref, cids_ref, cm_ref):
    b = pl.program_id(0)
    qm = q_ref[...]
    kb = k_ref[...]
    s = lax.dot_general(
        kb, qm, (((1,), (1,)), ((), ())),
        preferred_element_type=jnp.float32)
    row = lax.broadcasted_iota(jnp.int32, (BLK, Q), 0)
    s = jnp.where(row + b * BLK < K_REAL, s, NEG)

    for c in range(CPB):
        scores_ref[c] = s[c * CHUNK:(c + 1) * CHUNK, :].T

    cm = jnp.concatenate(
        [jnp.max(s[c * CHUNK:(c + 1) * CHUNK, :], axis=0, keepdims=True)
         for c in range(CPB)], axis=0)
    cm_ref[pl.ds(b * CPB, CPB), :] = cm

    @pl.when(b == NBLK - 1)
    def _pick_chunks():
        cmv = cm_ref[...]
        crow = lax.broadcasted_iota(jnp.int32, (NCHUNK, Q), 0)
        ids = []
        for _ in range(NDOCS):
            m = jnp.max(cmv, axis=0, keepdims=True)
            al = jnp.min(jnp.where(cmv == m, crow, NCHUNK), axis=0,
                         keepdims=True)
            ids.append(al)
            cmv = jnp.where(crow == al, NEG, cmv)
        packed = jnp.concatenate(
            ids + [jnp.zeros((16 - NDOCS, Q), jnp.int32)], axis=0)
        splats = [jnp.broadcast_to(ids[r], (16, Q)) for r in range(NDOCS)]
        cids_ref[...] = jnp.concatenate([packed] + splats, axis=0).T


def _sc_kernel(scores_hbm, cids_hbm, keys_hbm, vals_hbm, idx_hbm, sel_hbm,
               cids_v, gidx_a, gidx_b, buf_a, buf_b, ovals_v, oidx_v,
               selidx_v, selvec_v, sem_a, sem_b):
    wid = lax.axis_index("s") * 2 + lax.axis_index("c")
    base = wid * QPW
    pltpu.sync_copy(cids_hbm.at[pl.ds(base, QPW)], cids_v)
    iota16 = lax.iota(jnp.int32, 16)

    def fire(qi, gref, bref, sem):
        gref[...] = cids_v[qi, 0:16] * Q + (base + qi)
        pltpu.async_copy(scores_hbm.at[gref], bref, sem)

    def drain(gref, bref, sem):
        pltpu.make_async_copy(scores_hbm.at[gref], bref, sem).wait()

    def process(qi, bref):
        cand_v = jnp.full((16,), NEG, jnp.float32)
        cand_i = jnp.zeros((16,), jnp.int32)
        for r in range(NDOCS):
            csplat = cids_v[qi, 16 + r * 16:32 + r * 16]
            cbase = csplat * CHUNK
            for v in range(CHUNK // 16):
                vv = bref[r, v * 16:(v + 1) * 16]
                gi = cbase + v * 16 + iota16
                sv, si = plsc.sort_key_val(vv, gi, descending=True)
                take = sv > cand_v
                cand_v, cand_i = plsc.sort_key_val(
                    jnp.where(take, sv, cand_v),
                    jnp.where(take, si, cand_i))
        rv = lax.rev(cand_v, (0,))
        ri = lax.rev(cand_i, (0,))
        ovals_v[qi, :] = rv
        oidx_v[qi, :] = ri
        plsc.store_scatter(selidx_v, [jnp.full((16,), qi, jnp.int32)], ri,
                           mask=iota16 == 0)

    fire(0, gidx_a, buf_a, sem_a)

    def pair(i, carry):
        q0 = 2 * i
        q1 = q0 + 1
        fire(q1, gidx_b, buf_b, sem_b)
        drain(gidx_a, buf_a, sem_a)
        process(q0, buf_a)
        fire(jnp.minimum(q0 + 2, QPW - 1), gidx_a, buf_a, sem_a)
        drain(gidx_b, buf_b, sem_b)
        process(q1, buf_b)
        return carry

    lax.fori_loop(0, QPW // 2, pair, 0)
    drain(gidx_a, buf_a, sem_a)

    pltpu.sync_copy(ovals_v, vals_hbm.at[pl.ds(base, QPW)])
    pltpu.sync_copy(oidx_v, idx_hbm.at[pl.ds(base, QPW)])
    pltpu.async_copy(keys_hbm.at[selidx_v], selvec_v, sem_a).wait()
    pltpu.sync_copy(selvec_v, sel_hbm.at[pl.ds(base, QPW)])


@jax.jit
def kernel(queries, keys):
    keys_p = jnp.pad(keys, ((0, K_PAD - K_REAL), (0, 0)))

    scores, cids = pl.pallas_call(
        _tc_kernel,
        grid=(NBLK,),
        in_specs=[
            pl.BlockSpec((Q, D), lambda b: (0, 0)),
            pl.BlockSpec((BLK, D), lambda b: (b, 0)),
        ],
        out_specs=[
            pl.BlockSpec((CPB, Q, CHUNK), lambda b: (b, 0, 0)),
            pl.BlockSpec((Q, CID_W), lambda b: (0, 0)),
        ],
        out_shape=[
            jax.ShapeDtypeStruct((NCHUNK, Q, CHUNK), jnp.float32),
            jax.ShapeDtypeStruct((Q, CID_W), jnp.int32),
        ],
        scratch_shapes=[
            pltpu.VMEM((NCHUNK, Q), jnp.float32),
        ],
    )(queries, keys_p)

    scores2d = scores.reshape(NCHUNK * Q, CHUNK)

    mesh = plsc.VectorSubcoreMesh(core_axis_name="c", subcore_axis_name="s")
    keys128 = jnp.pad(keys_p, ((0, 0), (0, 128 - D)))

    vals, idx, sel = pl.kernel(
        _sc_kernel,
        mesh=mesh,
        compiler_params=pltpu.CompilerParams(needs_layout_passes=False),
        out_type=[
            jax.ShapeDtypeStruct((Q, 16), jnp.float32),
            jax.ShapeDtypeStruct((Q, 16), jnp.int32),
            jax.ShapeDtypeStruct((Q, 128), jnp.float32),
        ],
        scratch_types=[
            pltpu.VMEM((QPW, CID_W), jnp.int32),
            pltpu.VMEM((16,), jnp.int32),
            pltpu.VMEM((16,), jnp.int32),
            pltpu.VMEM((16, CHUNK), jnp.float32),
            pltpu.VMEM((16, CHUNK), jnp.float32),
            pltpu.VMEM((QPW, 16), jnp.float32),
            pltpu.VMEM((QPW, 16), jnp.int32),
            pltpu.VMEM((QPW,), jnp.int32),
            pltpu.VMEM((QPW, 128), jnp.float32),
            pltpu.SemaphoreType.DMA,
            pltpu.SemaphoreType.DMA,
        ],
    )(scores2d, cids, keys128)

    return vals[:, :NDOCS], idx[:, :NDOCS], sel[:, :D]
